# baseline (speedup 1.0000x reference)
import jax
import jax.numpy as jnp
from jax.experimental import pallas as pl

K = 50000
Q = 256
D = 6144
BLK = 1000
_EPS = 1e-12
_RTOL = 1e-5
_ATOL = 1e-8


def _prep_kernel(q_ref, t_ref, qnT_ref, thr_ref):
    q = q_ref[...]
    t = t_ref[...]
    qn = q / jnp.clip(jnp.sqrt(jnp.sum(q * q, axis=1, keepdims=True)), _EPS, None)
    tn = t / jnp.clip(jnp.sqrt(jnp.sum(t * t, axis=1, keepdims=True)), _EPS, None)
    qnT = qn.T
    tnT = tn.T
    qnT_ref[...] = qnT
    thr_ref[...] = jnp.sum(qnT * tnT, axis=0, keepdims=True)


def _count_kernel(data_ref, qnT_ref, thr_ref, out_ref):
    p = jnp.dot(data_ref[...], qnT_ref[...], preferred_element_type=jnp.float32)
    t = thr_ref[...]
    mask = jnp.logical_or(p >= t, jnp.abs(p - t) <= _ATOL + _RTOL * jnp.abs(t))
    partial = jnp.sum(mask.astype(jnp.int32), axis=0, keepdims=True)

    @pl.when(pl.program_id(0) == 0)
    def _():
        out_ref[...] = partial - 1

    @pl.when(pl.program_id(0) != 0)
    def _():
        out_ref[...] += partial


def kernel(queries, truths, data, query_idx_in_rankingset,
           use_actaul_mw_for_retrival, use_jaccard):
    qnT, thr = pl.pallas_call(
        _prep_kernel,
        out_shape=(
            jax.ShapeDtypeStruct((D, Q), jnp.float32),
            jax.ShapeDtypeStruct((1, Q), jnp.float32),
        ),
    )(queries, truths)

    ct = pl.pallas_call(
        _count_kernel,
        grid=(K // BLK,),
        in_specs=[
            pl.BlockSpec((BLK, D), lambda i: (i, 0)),
            pl.BlockSpec((D, Q), lambda i: (0, 0)),
            pl.BlockSpec((1, Q), lambda i: (0, 0)),
        ],
        out_specs=pl.BlockSpec((1, Q), lambda i: (0, 0)),
        out_shape=jax.ShapeDtypeStruct((1, Q), jnp.int32),
    )(data, qnT, thr)
    return ct


# speedup vs baseline: 1.0011x; 1.0011x over previous
import jax
import jax.numpy as jnp
from jax.experimental import pallas as pl

K = 50000
Q = 256
D = 6144
BLK = 1000
_EPS = 1e-12
_RTOL = 1e-5
_ATOL = 1e-8


def _prep_kernel(q_ref, t_ref, qnT_ref, thr_ref):
    q = q_ref[...]
    t = t_ref[...]
    qn = q / jnp.clip(jnp.sqrt(jnp.sum(q * q, axis=1, keepdims=True)), _EPS, None)
    tn = t / jnp.clip(jnp.sqrt(jnp.sum(t * t, axis=1, keepdims=True)), _EPS, None)
    qnT = qn.T
    tnT = tn.T
    qnT_ref[...] = qnT
    thr_ref[...] = jnp.sum(qnT * tnT, axis=0, keepdims=True)


def _count_kernel(data_ref, qnT_ref, thr_ref, out_ref):
    p = jnp.dot(data_ref[...].astype(jnp.bfloat16),
                qnT_ref[...].astype(jnp.bfloat16),
                preferred_element_type=jnp.float32)
    t = thr_ref[...]
    mask = jnp.logical_or(p >= t, jnp.abs(p - t) <= _ATOL + _RTOL * jnp.abs(t))
    partial = jnp.sum(mask.astype(jnp.int32), axis=0, keepdims=True)

    @pl.when(pl.program_id(0) == 0)
    def _():
        out_ref[...] = partial - 1

    @pl.when(pl.program_id(0) != 0)
    def _():
        out_ref[...] += partial


def kernel(queries, truths, data, query_idx_in_rankingset,
           use_actaul_mw_for_retrival, use_jaccard):
    qnT, thr = pl.pallas_call(
        _prep_kernel,
        out_shape=(
            jax.ShapeDtypeStruct((D, Q), jnp.float32),
            jax.ShapeDtypeStruct((1, Q), jnp.float32),
        ),
    )(queries, truths)

    ct = pl.pallas_call(
        _count_kernel,
        grid=(K // BLK,),
        in_specs=[
            pl.BlockSpec((BLK, D), lambda i: (i, 0)),
            pl.BlockSpec((D, Q), lambda i: (0, 0)),
            pl.BlockSpec((1, Q), lambda i: (0, 0)),
        ],
        out_specs=pl.BlockSpec((1, Q), lambda i: (0, 0)),
        out_shape=jax.ShapeDtypeStruct((1, Q), jnp.int32),
    )(data, qnT, thr)
    return ct


# speedup vs baseline: 1.0215x; 1.0204x over previous
import jax
import jax.numpy as jnp
from jax.experimental import pallas as pl
from jax.experimental.pallas import tpu as pltpu

K = 50000
Q = 256
D = 6144
BLK = 400
_EPS = 1e-12
_RTOL = 1e-5
_ATOL = 1e-8


def _count_kernel(data_ref, q_ref, t_ref, out_ref, qnT_s, thr_s):
    @pl.when(pl.program_id(0) == 0)
    def _prep():
        q = q_ref[...]
        t = t_ref[...]
        qn = q / jnp.clip(jnp.sqrt(jnp.sum(q * q, axis=1, keepdims=True)),
                          _EPS, None)
        tn = t / jnp.clip(jnp.sqrt(jnp.sum(t * t, axis=1, keepdims=True)),
                          _EPS, None)
        qnT = qn.T
        qnT_s[...] = qnT
        thr_s[...] = jnp.sum(qnT * tn.T, axis=0, keepdims=True)

    p = jnp.dot(data_ref[...], qnT_s[...], preferred_element_type=jnp.float32)
    t = thr_s[...]
    mask = jnp.logical_or(p >= t, jnp.abs(p - t) <= _ATOL + _RTOL * jnp.abs(t))
    partial = jnp.sum(mask.astype(jnp.int32), axis=0, keepdims=True)

    @pl.when(pl.program_id(0) == 0)
    def _():
        out_ref[...] = partial - 1

    @pl.when(pl.program_id(0) != 0)
    def _():
        out_ref[...] += partial


def kernel(queries, truths, data, query_idx_in_rankingset,
           use_actaul_mw_for_retrival, use_jaccard):
    return pl.pallas_call(
        _count_kernel,
        grid=(K // BLK,),
        in_specs=[
            pl.BlockSpec((BLK, D), lambda i: (i, 0)),
            pl.BlockSpec((Q, D), lambda i: (0, 0)),
            pl.BlockSpec((Q, D), lambda i: (0, 0)),
        ],
        out_specs=pl.BlockSpec((1, Q), lambda i: (0, 0)),
        out_shape=jax.ShapeDtypeStruct((1, Q), jnp.int32),
        scratch_shapes=[
            pltpu.VMEM((D, Q), jnp.float32),
            pltpu.VMEM((1, Q), jnp.float32),
        ],
    )(data, queries, truths)
